# baseline (speedup 1.0000x reference)
import functools

import jax
import jax.numpy as jnp
from jax import lax
from jax.experimental import pallas as pl
from jax.experimental.pallas import tpu as pltpu
from jax.experimental.pallas import tpu_sc as plsc

N = 4096
NUM_BINS = 30
LAMBD = 0.01

_RB_MM = 512

_NC = 2
_NS = 16
_NW = _NC * _NS
_TOTAL = N * N
_CHUNK = _TOTAL // _NW
_WIN = 16384
_NWIN = _CHUNK // _WIN
_UNROLL = 4


def _minmax_body(d_ref, o_ref, mn_ref, mx_ref):
    i = pl.program_id(0)

    @pl.when(i == 0)
    def _():
        mn_ref[0] = jnp.float32(jnp.inf)
        mx_ref[0] = jnp.float32(-jnp.inf)

    d = d_ref[...]
    mn_ref[0] = jnp.minimum(mn_ref[0], jnp.min(d))
    mx_ref[0] = jnp.maximum(mx_ref[0], jnp.max(d))

    @pl.when(i == pl.num_programs(0) - 1)
    def _():
        o_ref[0] = mn_ref[0]
        o_ref[1] = mx_ref[0]


_minmax = pl.pallas_call(
    _minmax_body,
    grid=(N // _RB_MM,),
    in_specs=[pl.BlockSpec((_RB_MM, N), lambda i: (i, 0))],
    out_specs=pl.BlockSpec(memory_space=pltpu.SMEM),
    out_shape=jax.ShapeDtypeStruct((2,), jnp.float32),
    scratch_shapes=[pltpu.SMEM((1,), jnp.float32),
                    pltpu.SMEM((1,), jnp.float32)],
)


def _sc_hist_body(w_hbm, d_hbm, bins_hbm, mn_hbm, ih_hbm, out_hbm,
                  wbuf, dbuf, binsbuf, mnbuf, ihbuf, hist):
    wid = lax.axis_index("s") * _NC + lax.axis_index("c")
    base = wid * _CHUNK

    pltpu.sync_copy(bins_hbm, binsbuf)
    pltpu.sync_copy(mn_hbm, mnbuf)
    pltpu.sync_copy(ih_hbm, ihbuf)

    for i in range(2 * _NS):
        hist[pl.ds(16 * i, 16)] = jnp.zeros((16,), jnp.float32)

    lane = lax.iota(jnp.int32, 16)
    mnv = mnbuf[...]
    ihv = ihbuf[...]

    def one_vec(off16):
        d = dbuf[pl.ds(off16, 16)]
        w = wbuf[pl.ds(off16, 16)]
        a = jnp.abs(w)
        t = (d - mnv) * ihv
        c = jnp.minimum((t + 1.0).astype(jnp.int32), 31)
        gdn = plsc.load_gather(binsbuf, [c - 1])
        k = c - jnp.where(gdn >= d, 1, 0)
        gup = plsc.load_gather(binsbuf, [k])
        k = k + jnp.where(gup < d, 1, 0)
        plsc.addupdate_scatter(hist, [lane * 32 + k], a)

    def win_body(wi, carry):
        off = base + wi * _WIN
        pltpu.sync_copy(d_hbm.at[pl.ds(off, _WIN)], dbuf)
        pltpu.sync_copy(w_hbm.at[pl.ds(off, _WIN)], wbuf)

        def inner(j, c2):
            for u in range(_UNROLL):
                one_vec(j * (16 * _UNROLL) + 16 * u)
            return c2

        lax.fori_loop(0, _WIN // (16 * _UNROLL), inner, 0)
        return carry

    lax.fori_loop(0, _NWIN, win_body, 0)
    pltpu.sync_copy(hist, out_hbm.at[wid])


_sc_hist = functools.partial(
    pl.kernel,
    out_type=jax.ShapeDtypeStruct((_NW, 2 * _NS * 16), jnp.float32),
    mesh=plsc.VectorSubcoreMesh(core_axis_name="c", subcore_axis_name="s",
                                num_cores=_NC, num_subcores=_NS),
    compiler_params=pltpu.CompilerParams(use_tc_tiling_on_sc=False,
                                         needs_layout_passes=False),
    scratch_types=[
        pltpu.VMEM((_WIN,), jnp.float32),
        pltpu.VMEM((_WIN,), jnp.float32),
        pltpu.VMEM((32,), jnp.float32),
        pltpu.VMEM((16,), jnp.float32),
        pltpu.VMEM((16,), jnp.float32),
        pltpu.VMEM((2 * _NS * 16,), jnp.float32),
    ],
)(_sc_hist_body)


def _combine_body(h_ref, o_ref):
    x = h_ref[...]
    s = jnp.sum(x, axis=0)
    t = s[1:NUM_BINS + 1]
    total = jnp.sum(t) + 1e-8
    p = t / total
    o_ref[0] = LAMBD * jnp.sum(p * jnp.log(p + 1e-8))


_combine = pl.pallas_call(
    _combine_body,
    out_specs=pl.BlockSpec(memory_space=pltpu.SMEM),
    out_shape=jax.ShapeDtypeStruct((1,), jnp.float32),
)


def kernel(weight_hh, distance_matrix):
    mm = _minmax(distance_matrix)
    mn, mx = mm[0], mm[1]
    bins = jnp.linspace(mn, mx, NUM_BINS + 1)
    bins32 = jnp.concatenate([bins, jnp.full((1,), jnp.inf, jnp.float32)])
    mn16 = jnp.full((16,), mn, jnp.float32)
    ih16 = jnp.full((16,), jnp.float32(NUM_BINS) / (mx - mn), jnp.float32)
    hist = _sc_hist(weight_hh.reshape(-1), distance_matrix.reshape(-1),
                    bins32, mn16, ih16)
    loss = _combine(hist.reshape(_NW * 16, 32))
    return loss[0]


# speedup vs baseline: 4.1564x; 4.1564x over previous
import functools

import jax
import jax.numpy as jnp
from jax import lax
from jax.experimental import pallas as pl
from jax.experimental.pallas import tpu as pltpu
from jax.experimental.pallas import tpu_sc as plsc

N = 4096
NUM_BINS = 30
LAMBD = 0.01

_RB_MM = 512

_NC = 2
_NS = 16
_NW = _NC * _NS
_WROWS = 4
_ROWS_PER_W = N // _NW
_NWIN = _ROWS_PER_W // _WROWS


def _minmax_body(d_ref, o_ref, mn_ref, mx_ref):
    i = pl.program_id(0)

    @pl.when(i == 0)
    def _():
        mn_ref[0] = jnp.float32(jnp.inf)
        mx_ref[0] = jnp.float32(-jnp.inf)

    d = d_ref[...]
    mn_ref[0] = jnp.minimum(mn_ref[0], jnp.min(d))
    mx_ref[0] = jnp.maximum(mx_ref[0], jnp.max(d))

    @pl.when(i == pl.num_programs(0) - 1)
    def _():
        o_ref[0] = mn_ref[0]
        o_ref[1] = mx_ref[0]


_minmax = pl.pallas_call(
    _minmax_body,
    grid=(N // _RB_MM,),
    in_specs=[pl.BlockSpec((_RB_MM, N), lambda i: (i, 0))],
    out_specs=pl.BlockSpec(memory_space=pltpu.SMEM),
    out_shape=jax.ShapeDtypeStruct((2,), jnp.float32),
    scratch_shapes=[pltpu.SMEM((1,), jnp.float32),
                    pltpu.SMEM((1,), jnp.float32)],
)


def _sc_hist_body(w_hbm, d_hbm, bins_hbm, mn_hbm, ih_hbm, out_hbm,
                  dbuf0, dbuf1, wbuf0, wbuf1, binsbuf, mnbuf, ihbuf, hist,
                  sd0, sd1, sw0, sw1):
    wid = lax.axis_index("s") * _NC + lax.axis_index("c")
    base_row = wid * _ROWS_PER_W

    pltpu.sync_copy(bins_hbm, binsbuf)
    pltpu.sync_copy(mn_hbm, mnbuf)
    pltpu.sync_copy(ih_hbm, ihbuf)

    for i in range(2 * _NS):
        hist[pl.ds(16 * i, 16)] = jnp.zeros((16,), jnp.float32)

    lane32 = lax.iota(jnp.int32, 16) * 32
    mnv = mnbuf[...]
    ihv = ihbuf[...]

    dbufs = (dbuf0, dbuf1)
    wbufs = (wbuf0, wbuf1)
    sds = (sd0, sd1)
    sws = (sw0, sw1)

    def start(win_idx, b):
        r0 = base_row + win_idx * _WROWS
        pltpu.async_copy(d_hbm.at[pl.ds(r0, _WROWS)], dbufs[b], sds[b])
        pltpu.async_copy(w_hbm.at[pl.ds(r0, _WROWS)], wbufs[b], sws[b])

    def wait(b):
        pltpu.make_async_copy(d_hbm.at[pl.ds(0, _WROWS)], dbufs[b],
                              sds[b]).wait()
        pltpu.make_async_copy(w_hbm.at[pl.ds(0, _WROWS)], wbufs[b],
                              sws[b]).wait()

    def compute(b):
        for r in range(_WROWS):
            dr = dbufs[b].at[r]
            wr = wbufs[b].at[r]

            @plsc.parallel_loop(0, N, step=16, unroll=8)
            def _(off):
                d = dr[pl.ds(off, 16)]
                w = wr[pl.ds(off, 16)]
                a = jnp.abs(w)
                t = (d - mnv) * ihv
                c = jnp.minimum(t.astype(jnp.int32), 30)
                g0 = plsc.load_gather(binsbuf, [c])
                g1 = plsc.load_gather(binsbuf, [c + 1])
                k = c + jnp.where(g0 < d, 1, 0) + jnp.where(g1 < d, 1, 0)
                plsc.addupdate_scatter(hist, [lane32 + k], a)

    start(0, 0)
    start(1, 1)

    def body2(h, carry):
        wait(0)
        compute(0)

        @pl.when(2 * h + 2 < _NWIN)
        def _():
            start(2 * h + 2, 0)

        wait(1)
        compute(1)

        @pl.when(2 * h + 3 < _NWIN)
        def _():
            start(2 * h + 3, 1)

        return carry

    lax.fori_loop(0, _NWIN // 2, body2, 0)
    pltpu.sync_copy(hist, out_hbm.at[wid])


_sc_hist = functools.partial(
    pl.kernel,
    out_type=jax.ShapeDtypeStruct((_NW, 2 * _NS * 16), jnp.float32),
    mesh=plsc.VectorSubcoreMesh(core_axis_name="c", subcore_axis_name="s",
                                num_cores=_NC, num_subcores=_NS),
    compiler_params=pltpu.CompilerParams(use_tc_tiling_on_sc=False,
                                         needs_layout_passes=False),
    scratch_types=[
        pltpu.VMEM((_WROWS, N), jnp.float32),
        pltpu.VMEM((_WROWS, N), jnp.float32),
        pltpu.VMEM((_WROWS, N), jnp.float32),
        pltpu.VMEM((_WROWS, N), jnp.float32),
        pltpu.VMEM((32,), jnp.float32),
        pltpu.VMEM((16,), jnp.float32),
        pltpu.VMEM((16,), jnp.float32),
        pltpu.VMEM((2 * _NS * 16,), jnp.float32),
        pltpu.SemaphoreType.DMA,
        pltpu.SemaphoreType.DMA,
        pltpu.SemaphoreType.DMA,
        pltpu.SemaphoreType.DMA,
    ],
)(_sc_hist_body)


def _combine_body(h_ref, o_ref):
    x = h_ref[...]
    s = jnp.sum(x, axis=0)
    t = s[1:NUM_BINS + 1]
    total = jnp.sum(t) + 1e-8
    p = t / total
    o_ref[0] = LAMBD * jnp.sum(p * jnp.log(p + 1e-8))


_combine = pl.pallas_call(
    _combine_body,
    out_specs=pl.BlockSpec(memory_space=pltpu.SMEM),
    out_shape=jax.ShapeDtypeStruct((1,), jnp.float32),
)


def kernel(weight_hh, distance_matrix):
    mm = _minmax(distance_matrix)
    mn, mx = mm[0], mm[1]
    bins = jnp.linspace(mn, mx, NUM_BINS + 1)
    bins32 = jnp.concatenate([bins, jnp.full((1,), jnp.inf, jnp.float32)])
    mn16 = jnp.full((16,), mn, jnp.float32)
    ih16 = jnp.full((16,), jnp.float32(NUM_BINS) / (mx - mn), jnp.float32)
    hist = _sc_hist(weight_hh, distance_matrix, bins32, mn16, ih16)
    loss = _combine(hist.reshape(_NW * 16, 32))
    return loss[0]


# speedup vs baseline: 5.7958x; 1.3944x over previous
import functools

import jax
import jax.numpy as jnp
from jax import lax
from jax.experimental import pallas as pl
from jax.experimental.pallas import tpu as pltpu
from jax.experimental.pallas import tpu_sc as plsc

N = 4096
NUM_BINS = 30
LAMBD = 0.01

_RB_MM = 512

_NC = 2
_NS = 16
_NW = _NC * _NS
_WROWS = 8
_WCOLS = 2048
_ROWS_PER_W = N // _NW
_NWIN = (_ROWS_PER_W // _WROWS) * (N // _WCOLS)


def _minmax_body(d_ref, o_ref, mn_ref, mx_ref):
    i = pl.program_id(0)

    @pl.when(i == 0)
    def _():
        mn_ref[0] = jnp.float32(jnp.inf)
        mx_ref[0] = jnp.float32(-jnp.inf)

    d = d_ref[...]
    mn_ref[0] = jnp.minimum(mn_ref[0], jnp.min(d))
    mx_ref[0] = jnp.maximum(mx_ref[0], jnp.max(d))

    @pl.when(i == pl.num_programs(0) - 1)
    def _():
        o_ref[0] = mn_ref[0]
        o_ref[1] = mx_ref[0]


_minmax = pl.pallas_call(
    _minmax_body,
    grid=(N // _RB_MM,),
    in_specs=[pl.BlockSpec((_RB_MM, N), lambda i: (i, 0))],
    out_specs=pl.BlockSpec(memory_space=pltpu.SMEM),
    out_shape=jax.ShapeDtypeStruct((2,), jnp.float32),
    scratch_shapes=[pltpu.SMEM((1,), jnp.float32),
                    pltpu.SMEM((1,), jnp.float32)],
)


def _sc_hist_body(w_hbm, d_hbm, bins_hbm, mn_hbm, ih_hbm, out_hbm,
                  dbuf0, dbuf1, wbuf0, wbuf1, binsbuf, mnbuf, ihbuf, hist,
                  sd0, sd1, sw0, sw1):
    wid = lax.axis_index("s") * _NC + lax.axis_index("c")
    base_row = wid * _ROWS_PER_W

    pltpu.sync_copy(bins_hbm, binsbuf)
    pltpu.sync_copy(mn_hbm, mnbuf)
    pltpu.sync_copy(ih_hbm, ihbuf)

    for i in range(2 * _NS):
        hist[pl.ds(16 * i, 16)] = jnp.zeros((16,), jnp.float32)

    lane32 = lax.iota(jnp.int32, 16) * 32
    mnv = mnbuf[...]
    ihv = ihbuf[...]

    dbufs = (dbuf0, dbuf1)
    wbufs = (wbuf0, wbuf1)
    sds = (sd0, sd1)
    sws = (sw0, sw1)

    def start(win_idx, b):
        r0 = base_row + (win_idx // 2) * _WROWS
        c0 = (win_idx % 2) * _WCOLS
        pltpu.async_copy(d_hbm.at[pl.ds(r0, _WROWS), pl.ds(c0, _WCOLS)],
                         dbufs[b], sds[b])
        pltpu.async_copy(w_hbm.at[pl.ds(r0, _WROWS), pl.ds(c0, _WCOLS)],
                         wbufs[b], sws[b])

    def wait(b):
        pltpu.make_async_copy(d_hbm.at[pl.ds(0, _WROWS), pl.ds(0, _WCOLS)],
                              dbufs[b], sds[b]).wait()
        pltpu.make_async_copy(w_hbm.at[pl.ds(0, _WROWS), pl.ds(0, _WCOLS)],
                              wbufs[b], sws[b]).wait()

    def compute(b):
        for r in range(_WROWS):
            db = dbufs[b]
            wb = wbufs[b]

            @plsc.parallel_loop(0, _WCOLS, step=16, unroll=8)
            def _(off):
                d = db[r, pl.ds(off, 16)]
                w = wb[r, pl.ds(off, 16)]
                a = jnp.abs(w)
                t = (d - mnv) * ihv
                c = jnp.minimum(t.astype(jnp.int32), 30)
                g0 = plsc.load_gather(binsbuf, [c])
                g1 = plsc.load_gather(binsbuf, [c + 1])
                k = c + jnp.where(g0 < d, 1, 0) + jnp.where(g1 < d, 1, 0)
                plsc.addupdate_scatter(hist, [lane32 + k], a)

    start(0, 0)
    start(1, 1)

    def body2(h, carry):
        wait(0)
        compute(0)

        @pl.when(2 * h + 2 < _NWIN)
        def _():
            start(2 * h + 2, 0)

        wait(1)
        compute(1)

        @pl.when(2 * h + 3 < _NWIN)
        def _():
            start(2 * h + 3, 1)

        return carry

    lax.fori_loop(0, _NWIN // 2, body2, 0)
    pltpu.sync_copy(hist, out_hbm.at[wid])


_sc_hist = functools.partial(
    pl.kernel,
    out_type=jax.ShapeDtypeStruct((_NW, 2 * _NS * 16), jnp.float32),
    mesh=plsc.VectorSubcoreMesh(core_axis_name="c", subcore_axis_name="s",
                                num_cores=_NC, num_subcores=_NS),
    compiler_params=pltpu.CompilerParams(use_tc_tiling_on_sc=True,
                                         needs_layout_passes=False),
    scratch_types=[
        pltpu.VMEM((_WROWS, _WCOLS), jnp.float32),
        pltpu.VMEM((_WROWS, _WCOLS), jnp.float32),
        pltpu.VMEM((_WROWS, _WCOLS), jnp.float32),
        pltpu.VMEM((_WROWS, _WCOLS), jnp.float32),
        pltpu.VMEM((32,), jnp.float32),
        pltpu.VMEM((16,), jnp.float32),
        pltpu.VMEM((16,), jnp.float32),
        pltpu.VMEM((2 * _NS * 16,), jnp.float32),
        pltpu.SemaphoreType.DMA,
        pltpu.SemaphoreType.DMA,
        pltpu.SemaphoreType.DMA,
        pltpu.SemaphoreType.DMA,
    ],
)(_sc_hist_body)


def _combine_body(h_ref, o_ref):
    x = h_ref[...]
    s = jnp.sum(x, axis=0)
    t = s[1:NUM_BINS + 1]
    total = jnp.sum(t) + 1e-8
    p = t / total
    o_ref[0] = LAMBD * jnp.sum(p * jnp.log(p + 1e-8))


_combine = pl.pallas_call(
    _combine_body,
    out_specs=pl.BlockSpec(memory_space=pltpu.SMEM),
    out_shape=jax.ShapeDtypeStruct((1,), jnp.float32),
)


def kernel(weight_hh, distance_matrix):
    mm = _minmax(distance_matrix)
    mn, mx = mm[0], mm[1]
    bins = jnp.linspace(mn, mx, NUM_BINS + 1)
    bins32 = jnp.concatenate([bins, jnp.full((1,), jnp.inf, jnp.float32)])
    mn16 = jnp.full((16,), mn, jnp.float32)
    ih16 = jnp.full((16,), jnp.float32(NUM_BINS) / (mx - mn), jnp.float32)
    hist = _sc_hist(weight_hh, distance_matrix, bins32, mn16, ih16)
    loss = _combine(hist.reshape(_NW * 16, 32))
    return loss[0]


# speedup vs baseline: 6.6639x; 1.1498x over previous
import functools

import jax
import jax.numpy as jnp
from jax import lax
from jax.experimental import pallas as pl
from jax.experimental.pallas import tpu as pltpu
from jax.experimental.pallas import tpu_sc as plsc

N = 4096
NUM_BINS = 30
LAMBD = 0.01

_RB_MM = 512

_NC = 2
_NS = 16
_NW = _NC * _NS
_WROWS = 8
_WCOLS = 2048
_ROWS_PER_W = N // _NW
_NWIN = (_ROWS_PER_W // _WROWS) * (N // _WCOLS)


def _minmax_body(d_ref, o_ref, mn_ref, mx_ref):
    i = pl.program_id(0)

    @pl.when(i == 0)
    def _():
        mn_ref[0] = jnp.float32(jnp.inf)
        mx_ref[0] = jnp.float32(-jnp.inf)

    d = d_ref[...]
    mn_ref[0] = jnp.minimum(mn_ref[0], jnp.min(d))
    mx_ref[0] = jnp.maximum(mx_ref[0], jnp.max(d))

    @pl.when(i == pl.num_programs(0) - 1)
    def _():
        o_ref[0] = mn_ref[0]
        o_ref[1] = mx_ref[0]


_minmax = pl.pallas_call(
    _minmax_body,
    grid=(N // _RB_MM,),
    in_specs=[pl.BlockSpec((_RB_MM, N), lambda i: (i, 0))],
    out_specs=pl.BlockSpec(memory_space=pltpu.SMEM),
    out_shape=jax.ShapeDtypeStruct((2,), jnp.float32),
    scratch_shapes=[pltpu.SMEM((1,), jnp.float32),
                    pltpu.SMEM((1,), jnp.float32)],
)


def _sc_hist_body(w_hbm, d_hbm, bins_hbm, bias_hbm, ih_hbm, out_hbm,
                  dbuf0, dbuf1, wbuf0, wbuf1, binsbuf, biasbuf, ihbuf, hist,
                  sd0, sd1, sw0, sw1):
    wid = lax.axis_index("s") * _NC + lax.axis_index("c")
    base_row = wid * _ROWS_PER_W

    pltpu.sync_copy(bins_hbm, binsbuf)
    pltpu.sync_copy(bias_hbm, biasbuf)
    pltpu.sync_copy(ih_hbm, ihbuf)

    for i in range(2 * _NS):
        hist[pl.ds(16 * i, 16)] = jnp.zeros((16,), jnp.float32)

    lane32 = lax.iota(jnp.int32, 16) * 32
    bv = biasbuf[...]
    ihv = ihbuf[...]

    dbufs = (dbuf0, dbuf1)
    wbufs = (wbuf0, wbuf1)
    sds = (sd0, sd1)
    sws = (sw0, sw1)

    def start(win_idx, b):
        r0 = base_row + (win_idx // 2) * _WROWS
        c0 = (win_idx % 2) * _WCOLS
        pltpu.async_copy(d_hbm.at[pl.ds(r0, _WROWS), pl.ds(c0, _WCOLS)],
                         dbufs[b], sds[b])
        pltpu.async_copy(w_hbm.at[pl.ds(r0, _WROWS), pl.ds(c0, _WCOLS)],
                         wbufs[b], sws[b])

    def wait(b):
        pltpu.make_async_copy(d_hbm.at[pl.ds(0, _WROWS), pl.ds(0, _WCOLS)],
                              dbufs[b], sds[b]).wait()
        pltpu.make_async_copy(w_hbm.at[pl.ds(0, _WROWS), pl.ds(0, _WCOLS)],
                              wbufs[b], sws[b]).wait()

    def compute(b):
        for r in range(_WROWS):
            db = dbufs[b]
            wb = wbufs[b]

            @plsc.parallel_loop(0, _WCOLS, step=16, unroll=8)
            def _(off):
                d = db[r, pl.ds(off, 16)]
                w = wb[r, pl.ds(off, 16)]
                a = jnp.abs(w)
                t = d * ihv + bv
                c = t.astype(jnp.int32)
                g = plsc.load_gather(binsbuf, [c])
                k = c + jnp.where(g < d, 1, 0)
                plsc.addupdate_scatter(hist, [lane32 + k], a)

    start(0, 0)
    start(1, 1)

    def body2(h, carry):
        wait(0)
        compute(0)

        @pl.when(2 * h + 2 < _NWIN)
        def _():
            start(2 * h + 2, 0)

        wait(1)
        compute(1)

        @pl.when(2 * h + 3 < _NWIN)
        def _():
            start(2 * h + 3, 1)

        return carry

    lax.fori_loop(0, _NWIN // 2, body2, 0)
    pltpu.sync_copy(hist, out_hbm.at[wid])


_sc_hist = functools.partial(
    pl.kernel,
    out_type=jax.ShapeDtypeStruct((_NW, 2 * _NS * 16), jnp.float32),
    mesh=plsc.VectorSubcoreMesh(core_axis_name="c", subcore_axis_name="s",
                                num_cores=_NC, num_subcores=_NS),
    compiler_params=pltpu.CompilerParams(use_tc_tiling_on_sc=True,
                                         needs_layout_passes=False),
    scratch_types=[
        pltpu.VMEM((_WROWS, _WCOLS), jnp.float32),
        pltpu.VMEM((_WROWS, _WCOLS), jnp.float32),
        pltpu.VMEM((_WROWS, _WCOLS), jnp.float32),
        pltpu.VMEM((_WROWS, _WCOLS), jnp.float32),
        pltpu.VMEM((32,), jnp.float32),
        pltpu.VMEM((16,), jnp.float32),
        pltpu.VMEM((16,), jnp.float32),
        pltpu.VMEM((2 * _NS * 16,), jnp.float32),
        pltpu.SemaphoreType.DMA,
        pltpu.SemaphoreType.DMA,
        pltpu.SemaphoreType.DMA,
        pltpu.SemaphoreType.DMA,
    ],
)(_sc_hist_body)


def _combine_body(h_ref, o_ref):
    x = h_ref[...]
    s = jnp.sum(x, axis=0)
    t = s[1:NUM_BINS + 1]
    total = jnp.sum(t) + 1e-8
    p = t / total
    o_ref[0] = LAMBD * jnp.sum(p * jnp.log(p + 1e-8))


_combine = pl.pallas_call(
    _combine_body,
    out_specs=pl.BlockSpec(memory_space=pltpu.SMEM),
    out_shape=jax.ShapeDtypeStruct((1,), jnp.float32),
)


def kernel(weight_hh, distance_matrix):
    mm = _minmax(distance_matrix)
    mn, mx = mm[0], mm[1]
    bins = jnp.linspace(mn, mx, NUM_BINS + 1)
    bins32 = jnp.concatenate([bins, jnp.full((1,), jnp.inf, jnp.float32)])
    ih_up = (jnp.float32(NUM_BINS) / (mx - mn)) * jnp.float32(1.0 + 1e-4)
    bias16 = jnp.full((16,), jnp.float32(0.01) - mn * ih_up, jnp.float32)
    ih16 = jnp.full((16,), ih_up, jnp.float32)
    hist = _sc_hist(weight_hh, distance_matrix, bins32, bias16, ih16)
    loss = _combine(hist.reshape(_NW * 16, 32))
    return loss[0]


# speedup vs baseline: 7.6445x; 1.1472x over previous
import functools

import jax
import jax.numpy as jnp
from jax import lax
from jax.experimental import pallas as pl
from jax.experimental.pallas import tpu as pltpu
from jax.experimental.pallas import tpu_sc as plsc

N = 4096
NUM_BINS = 30
LAMBD = 0.01

_RB_MM = 512

_NC = 2
_NS = 16
_NW = _NC * _NS
_WROWS = 8
_WCOLS = 2048
_ROWS_PER_W = N // _NW
_NWIN = (_ROWS_PER_W // _WROWS) * (N // _WCOLS)


def _minmax_body(d_ref, o_ref, mn_ref, mx_ref):
    i = pl.program_id(0)

    @pl.when(i == 0)
    def _():
        mn_ref[0] = jnp.float32(jnp.inf)
        mx_ref[0] = jnp.float32(-jnp.inf)

    d = d_ref[...]
    mn_ref[0] = jnp.minimum(mn_ref[0], jnp.min(d))
    mx_ref[0] = jnp.maximum(mx_ref[0], jnp.max(d))

    @pl.when(i == pl.num_programs(0) - 1)
    def _():
        o_ref[0] = mn_ref[0]
        o_ref[1] = mx_ref[0]


_minmax = pl.pallas_call(
    _minmax_body,
    grid=(N // _RB_MM,),
    in_specs=[pl.BlockSpec((_RB_MM, N), lambda i: (i, 0))],
    out_specs=pl.BlockSpec(memory_space=pltpu.SMEM),
    out_shape=jax.ShapeDtypeStruct((2,), jnp.float32),
    scratch_shapes=[pltpu.SMEM((1,), jnp.float32),
                    pltpu.SMEM((1,), jnp.float32)],
)


def _sc_hist_body(w_hbm, d_hbm, bins_hbm, bias_hbm, ih_hbm, out_hbm,
                  dbuf0, dbuf1, wbuf0, wbuf1, binsbuf, biasbuf, ihbuf, hist,
                  sd0, sd1, sw0, sw1):
    wid = lax.axis_index("s") * _NC + lax.axis_index("c")
    base_row = wid * _ROWS_PER_W

    pltpu.sync_copy(bins_hbm, binsbuf)
    pltpu.sync_copy(bias_hbm, biasbuf)
    pltpu.sync_copy(ih_hbm, ihbuf)

    for i in range(2 * _NS):
        hist[i, :] = jnp.zeros((16,), jnp.float32)

    lane = lax.iota(jnp.int32, 16)
    bv = biasbuf[...]
    ihv = ihbuf[...]

    dbufs = (dbuf0, dbuf1)
    wbufs = (wbuf0, wbuf1)
    sds = (sd0, sd1)
    sws = (sw0, sw1)

    def start(win_idx, b):
        r0 = base_row + (win_idx // 2) * _WROWS
        c0 = (win_idx % 2) * _WCOLS
        pltpu.async_copy(d_hbm.at[pl.ds(r0, _WROWS), pl.ds(c0, _WCOLS)],
                         dbufs[b], sds[b])
        pltpu.async_copy(w_hbm.at[pl.ds(r0, _WROWS), pl.ds(c0, _WCOLS)],
                         wbufs[b], sws[b])

    def wait(b):
        pltpu.make_async_copy(d_hbm.at[pl.ds(0, _WROWS), pl.ds(0, _WCOLS)],
                              dbufs[b], sds[b]).wait()
        pltpu.make_async_copy(w_hbm.at[pl.ds(0, _WROWS), pl.ds(0, _WCOLS)],
                              wbufs[b], sws[b]).wait()

    def compute(b):
        for r in range(_WROWS):
            db = dbufs[b]
            wb = wbufs[b]

            @plsc.parallel_loop(0, _WCOLS, step=16, unroll=8)
            def _(off):
                d = db[r, pl.ds(off, 16)]
                w = wb[r, pl.ds(off, 16)]
                a = jnp.abs(w)
                t = d * ihv + bv
                c = t.astype(jnp.int32)
                g = plsc.load_gather(binsbuf, [c])
                k = c + jnp.where(g < d, 1, 0)
                plsc.addupdate_scatter(hist, [k, lane], a)

    start(0, 0)
    start(1, 1)

    def body2(h, carry):
        wait(0)
        compute(0)

        @pl.when(2 * h + 2 < _NWIN)
        def _():
            start(2 * h + 2, 0)

        wait(1)
        compute(1)

        @pl.when(2 * h + 3 < _NWIN)
        def _():
            start(2 * h + 3, 1)

        return carry

    lax.fori_loop(0, _NWIN // 2, body2, 0)
    pltpu.sync_copy(hist, out_hbm.at[wid])


_sc_hist = functools.partial(
    pl.kernel,
    out_type=jax.ShapeDtypeStruct((_NW, 2 * _NS, 16), jnp.float32),
    mesh=plsc.VectorSubcoreMesh(core_axis_name="c", subcore_axis_name="s",
                                num_cores=_NC, num_subcores=_NS),
    compiler_params=pltpu.CompilerParams(use_tc_tiling_on_sc=True,
                                         needs_layout_passes=False),
    scratch_types=[
        pltpu.VMEM((_WROWS, _WCOLS), jnp.float32),
        pltpu.VMEM((_WROWS, _WCOLS), jnp.float32),
        pltpu.VMEM((_WROWS, _WCOLS), jnp.float32),
        pltpu.VMEM((_WROWS, _WCOLS), jnp.float32),
        pltpu.VMEM((32,), jnp.float32),
        pltpu.VMEM((16,), jnp.float32),
        pltpu.VMEM((16,), jnp.float32),
        pltpu.VMEM((2 * _NS, 16), jnp.float32),
        pltpu.SemaphoreType.DMA,
        pltpu.SemaphoreType.DMA,
        pltpu.SemaphoreType.DMA,
        pltpu.SemaphoreType.DMA,
    ],
)(_sc_hist_body)


def _combine_body(h_ref, o_ref):
    x = h_ref[...]
    s = jnp.sum(jnp.sum(x, axis=2), axis=0)
    t = s[1:NUM_BINS + 1]
    total = jnp.sum(t) + 1e-8
    p = t / total
    o_ref[0] = LAMBD * jnp.sum(p * jnp.log(p + 1e-8))


_combine = pl.pallas_call(
    _combine_body,
    out_specs=pl.BlockSpec(memory_space=pltpu.SMEM),
    out_shape=jax.ShapeDtypeStruct((1,), jnp.float32),
)


def kernel(weight_hh, distance_matrix):
    mm = _minmax(distance_matrix)
    mn, mx = mm[0], mm[1]
    bins = jnp.linspace(mn, mx, NUM_BINS + 1)
    bins32 = jnp.concatenate([bins, jnp.full((1,), jnp.inf, jnp.float32)])
    ih_up = (jnp.float32(NUM_BINS) / (mx - mn)) * jnp.float32(1.0 + 1e-4)
    bias16 = jnp.full((16,), jnp.float32(0.01) - mn * ih_up, jnp.float32)
    ih16 = jnp.full((16,), ih_up, jnp.float32)
    hist = _sc_hist(weight_hh, distance_matrix, bins32, bias16, ih16)
    loss = _combine(hist)
    return loss[0]


# speedup vs baseline: 7.7354x; 1.0119x over previous
import functools

import jax
import jax.numpy as jnp
from jax import lax
from jax.experimental import pallas as pl
from jax.experimental.pallas import tpu as pltpu
from jax.experimental.pallas import tpu_sc as plsc

N = 4096
NUM_BINS = 30
LAMBD = 0.01

_RB_MM = 512

_NC = 2
_NS = 16
_NW = _NC * _NS
_WROWS = 8
_WCOLS = 2048
_ROWS_PER_W = N // _NW
_NWIN = (_ROWS_PER_W // _WROWS) * (N // _WCOLS)


def _minmax_body(d_ref, bins_ref, bias_ref, ih_ref, mn_ref, mx_ref):
    i = pl.program_id(0)

    @pl.when(i == 0)
    def _():
        mn_ref[0] = jnp.float32(jnp.inf)
        mx_ref[0] = jnp.float32(-jnp.inf)

    d = d_ref[...]
    mn_ref[0] = jnp.minimum(mn_ref[0], jnp.min(d))
    mx_ref[0] = jnp.maximum(mx_ref[0], jnp.max(d))

    @pl.when(i == pl.num_programs(0) - 1)
    def _():
        mn = mn_ref[0]
        mx = mx_ref[0]
        f = lax.broadcasted_iota(jnp.int32, (1, NUM_BINS), 1).astype(
            jnp.float32) / NUM_BINS
        lo = mn * (1.0 - f) + mx * f
        bins_ref[...] = jnp.concatenate(
            [lo, jnp.full((1, 1), mx, jnp.float32),
             jnp.full((1, 1), jnp.inf, jnp.float32)], axis=1)
        ih_up = (jnp.float32(NUM_BINS) / (mx - mn)) * jnp.float32(1.0 + 1e-4)
        ih_ref[...] = jnp.full((1, 16), ih_up, jnp.float32)
        bias_ref[...] = jnp.full((1, 16), jnp.float32(0.01) - mn * ih_up,
                                 jnp.float32)


_minmax = pl.pallas_call(
    _minmax_body,
    grid=(N // _RB_MM,),
    in_specs=[pl.BlockSpec((_RB_MM, N), lambda i: (i, 0))],
    out_specs=[pl.BlockSpec((1, 32), lambda i: (0, 0)),
               pl.BlockSpec((1, 16), lambda i: (0, 0)),
               pl.BlockSpec((1, 16), lambda i: (0, 0))],
    out_shape=[jax.ShapeDtypeStruct((1, 32), jnp.float32),
               jax.ShapeDtypeStruct((1, 16), jnp.float32),
               jax.ShapeDtypeStruct((1, 16), jnp.float32)],
    scratch_shapes=[pltpu.SMEM((1,), jnp.float32),
                    pltpu.SMEM((1,), jnp.float32)],
)


def _sc_hist_body(w_hbm, d_hbm, bins_hbm, bias_hbm, ih_hbm, out_hbm,
                  dbuf0, dbuf1, wbuf0, wbuf1, binsbuf, biasbuf, ihbuf, hist,
                  sd0, sd1, sw0, sw1):
    wid = lax.axis_index("s") * _NC + lax.axis_index("c")
    base_row = wid * _ROWS_PER_W

    pltpu.sync_copy(bins_hbm, binsbuf)
    pltpu.sync_copy(bias_hbm, biasbuf)
    pltpu.sync_copy(ih_hbm, ihbuf)

    for i in range(2 * _NS):
        hist[i, :] = jnp.zeros((16,), jnp.float32)

    lane = lax.iota(jnp.int32, 16)
    zero16 = jnp.zeros((16,), jnp.int32)
    bv = biasbuf[0, :]
    ihv = ihbuf[0, :]

    dbufs = (dbuf0, dbuf1)
    wbufs = (wbuf0, wbuf1)
    sds = (sd0, sd1)
    sws = (sw0, sw1)

    def start(win_idx, b):
        r0 = base_row + (win_idx // 2) * _WROWS
        c0 = (win_idx % 2) * _WCOLS
        pltpu.async_copy(d_hbm.at[pl.ds(r0, _WROWS), pl.ds(c0, _WCOLS)],
                         dbufs[b], sds[b])
        pltpu.async_copy(w_hbm.at[pl.ds(r0, _WROWS), pl.ds(c0, _WCOLS)],
                         wbufs[b], sws[b])

    def wait(b):
        pltpu.make_async_copy(d_hbm.at[pl.ds(0, _WROWS), pl.ds(0, _WCOLS)],
                              dbufs[b], sds[b]).wait()
        pltpu.make_async_copy(w_hbm.at[pl.ds(0, _WROWS), pl.ds(0, _WCOLS)],
                              wbufs[b], sws[b]).wait()

    def compute(b):
        for r in range(_WROWS):
            db = dbufs[b]
            wb = wbufs[b]

            @plsc.parallel_loop(0, _WCOLS, step=16, unroll=16)
            def _(off):
                d = db[r, pl.ds(off, 16)]
                w = wb[r, pl.ds(off, 16)]
                a = jnp.abs(w)
                t = d * ihv + bv
                c = t.astype(jnp.int32)
                g = plsc.load_gather(binsbuf, [zero16, c])
                k = c + jnp.where(g < d, 1, 0)
                plsc.addupdate_scatter(hist, [k, lane], a)

    start(0, 0)
    start(1, 1)

    def body2(h, carry):
        wait(0)
        compute(0)

        @pl.when(2 * h + 2 < _NWIN)
        def _():
            start(2 * h + 2, 0)

        wait(1)
        compute(1)

        @pl.when(2 * h + 3 < _NWIN)
        def _():
            start(2 * h + 3, 1)

        return carry

    lax.fori_loop(0, _NWIN // 2, body2, 0)
    pltpu.sync_copy(hist, out_hbm.at[wid])


_sc_hist = functools.partial(
    pl.kernel,
    out_type=jax.ShapeDtypeStruct((_NW, 2 * _NS, 16), jnp.float32),
    mesh=plsc.VectorSubcoreMesh(core_axis_name="c", subcore_axis_name="s",
                                num_cores=_NC, num_subcores=_NS),
    compiler_params=pltpu.CompilerParams(use_tc_tiling_on_sc=True,
                                         needs_layout_passes=False),
    scratch_types=[
        pltpu.VMEM((_WROWS, _WCOLS), jnp.float32),
        pltpu.VMEM((_WROWS, _WCOLS), jnp.float32),
        pltpu.VMEM((_WROWS, _WCOLS), jnp.float32),
        pltpu.VMEM((_WROWS, _WCOLS), jnp.float32),
        pltpu.VMEM((1, 32), jnp.float32),
        pltpu.VMEM((1, 16), jnp.float32),
        pltpu.VMEM((1, 16), jnp.float32),
        pltpu.VMEM((2 * _NS, 16), jnp.float32),
        pltpu.SemaphoreType.DMA,
        pltpu.SemaphoreType.DMA,
        pltpu.SemaphoreType.DMA,
        pltpu.SemaphoreType.DMA,
    ],
)(_sc_hist_body)


def _combine_body(h_ref, o_ref):
    x = h_ref[...]
    s = jnp.sum(jnp.sum(x, axis=2), axis=0)
    t = s[1:NUM_BINS + 1]
    total = jnp.sum(t) + 1e-8
    p = t / total
    o_ref[0] = LAMBD * jnp.sum(p * jnp.log(p + 1e-8))


_combine = pl.pallas_call(
    _combine_body,
    out_specs=pl.BlockSpec(memory_space=pltpu.SMEM),
    out_shape=jax.ShapeDtypeStruct((1,), jnp.float32),
)


def kernel(weight_hh, distance_matrix):
    bins32, bias16, ih16 = _minmax(distance_matrix)
    hist = _sc_hist(weight_hh, distance_matrix, bins32, bias16, ih16)
    loss = _combine(hist)
    return loss[0]
